# Initial kernel scaffold; baseline (speedup 1.0000x reference)
import jax
import jax.numpy as jnp
from jax.experimental import pallas as pl


def kernel(inp, input1):
    raise NotImplementedError("write your pallas kernel here")


# speedup vs baseline: 9.2805x; 9.2805x over previous
import functools

import jax
import jax.numpy as jnp
from jax import lax
from jax.experimental import pallas as pl
from jax.experimental.pallas import tpu as pltpu
from jax.experimental.pallas import tpu_sc as plsc

B = 16
P = 30
J = 17
N = 16384
L = 16
PW = 2 * L
NC = 2


def _splat_i32(value):
    return jnp.full((L,), value, dtype=jnp.int32)


_mesh = plsc.VectorSubcoreMesh(core_axis_name="c", subcore_axis_name="s")


@functools.partial(
    pl.kernel,
    mesh=_mesh,
    compiler_params=pltpu.CompilerParams(needs_layout_passes=False),
    out_type=jax.ShapeDtypeStruct((B, L), jnp.float32),
    scratch_types=[
        pltpu.VMEM((N,), jnp.float32),
        pltpu.VMEM((P * J * 2 + 4,), jnp.int32),
        pltpu.VMEM((J, PW), jnp.float32),
        pltpu.VMEM((J, PW), jnp.float32),
        pltpu.VMEM((PW,), jnp.float32),
        pltpu.VMEM((L,), jnp.float32),
    ],
)
def _aeloss_sc(tags_hbm, kp_hbm, out_hbm, tags_v, kp_v, vt_v, fm_v, m_v, o_v):
    wid = lax.axis_index("s") * NC + lax.axis_index("c")

    @pl.when(wid < B)
    def _body():
        b = wid
        pltpu.sync_copy(tags_hbm.at[b], tags_v)
        pltpu.sync_copy(kp_hbm.at[b], kp_v)

        lane = lax.iota(jnp.int32, L)
        pids = (lane, lane + L)
        pclamp = tuple(jnp.minimum(p, P - 1) for p in pids)
        pvalidlane = tuple(p < P for p in pids)

        acc = [jnp.zeros((L,), jnp.float32), jnp.zeros((L,), jnp.float32)]
        cnt = [jnp.zeros((L,), jnp.float32), jnp.zeros((L,), jnp.float32)]
        for j in range(J):
            for h in range(2):
                base = pclamp[h] * (J * 2) + (j * 2)
                idx = plsc.load_gather(kp_v, [base])
                flag = plsc.load_gather(kp_v, [base + 1])
                fm = jnp.where(pvalidlane[h] & (flag == 1), 1.0, 0.0)
                vt = plsc.load_gather(
                    tags_v, [jnp.clip(idx, 0, N - 1).astype(jnp.int32)]
                )
                vt_v[j, pl.ds(h * L, L)] = vt
                fm_v[j, pl.ds(h * L, L)] = fm
                acc[h] = acc[h] + vt * fm
                cnt[h] = cnt[h] + fm

        m = []
        safe = []
        validf = []
        for h in range(2):
            sc = jnp.maximum(cnt[h], 1.0)
            vf = jnp.where(cnt[h] > 0.0, 1.0, 0.0)
            mh = vf * (acc[h] / sc)
            m_v[pl.ds(h * L, L)] = mh
            m.append(mh)
            safe.append(sc)
            validf.append(vf)
        curv = jnp.full((L,), jnp.sum(validf[0] + validf[1]), dtype=jnp.float32)

        pull_parts = []
        for h in range(2):
            pp = jnp.zeros((L,), jnp.float32)
            for j in range(J):
                vt = vt_v[j, pl.ds(h * L, L)]
                fm = fm_v[j, pl.ds(h * L, L)]
                d = vt - m[h]
                pp = pp + d * d * fm
            pull_parts.append(validf[h] * (pp / safe[h]))
        pullv = jnp.full(
            (L,), jnp.sum(pull_parts[0] + pull_parts[1]), dtype=jnp.float32
        )
        pullv = jnp.where(curv > 0.0, pullv / curv, pullv)

        lanef = (lane.astype(jnp.float32), (lane + L).astype(jnp.float32))
        s = [jnp.zeros((L,), jnp.float32), jnp.zeros((L,), jnp.float32)]
        for i in range(P):
            mi = plsc.load_gather(m_v, [_splat_i32(i)])
            for h in range(2):
                d = mi - m[h]
                e = jnp.exp(-(d * d))
                keep = (pids[h] > i) & (lanef[h] < curv)
                s[h] = s[h] + jnp.where(keep, e, 0.0)
        pushv = jnp.full((L,), jnp.sum(s[0] + s[1]), dtype=jnp.float32)
        denomv = jnp.maximum(curv * (curv - 1.0), 1.0) * 0.5
        pushv = jnp.where(curv > 1.0, pushv / denomv, pushv) * 0.5

        o_v[...] = jnp.where(
            lane == 0, pushv, jnp.where(lane == 1, pullv, 0.0)
        ).astype(jnp.float32)
        pltpu.sync_copy(o_v, out_hbm.at[b])


def kernel(inp, input1):
    tags = inp.reshape(B, N)
    kp = input1.astype(jnp.int32).reshape(B, P * J * 2)
    kp = jnp.pad(kp, ((0, 0), (0, 4)))
    out = _aeloss_sc(tags, kp)
    return out[:, :2]


# speedup vs baseline: 9.2978x; 1.0019x over previous
import functools

import jax
import jax.numpy as jnp
from jax import lax
from jax.experimental import pallas as pl
from jax.experimental.pallas import tpu as pltpu
from jax.experimental.pallas import tpu_sc as plsc

B = 16
P = 30
J = 17
N = 16384
L = 16
NC = 2


def _splat_i32(value):
    return jnp.full((L,), value, dtype=jnp.int32)


_mesh = plsc.VectorSubcoreMesh(core_axis_name="c", subcore_axis_name="s")


@functools.partial(
    pl.kernel,
    mesh=_mesh,
    compiler_params=pltpu.CompilerParams(needs_layout_passes=False),
    out_type=jax.ShapeDtypeStruct((B, L), jnp.float32),
    scratch_types=[
        pltpu.VMEM((N,), jnp.float32),
        pltpu.VMEM((P * J * 2 + 4,), jnp.int32),
        pltpu.VMEM((J, 2 * L), jnp.float32),
        pltpu.VMEM((J, 2 * L), jnp.float32),
        pltpu.VMEM((2 * L,), jnp.float32),
        pltpu.VMEM((L,), jnp.float32),
    ],
)
def _aeloss_sc(tags_hbm, kp_hbm, out_hbm, tags_v, kp_v, vt_v, fm_v, m_v, o_v):
    wid = lax.axis_index("s") * NC + lax.axis_index("c")

    @pl.when(wid < B)
    def _body():
        b = wid
        pltpu.sync_copy(tags_hbm.at[b], tags_v)
        pltpu.sync_copy(kp_hbm.at[b], kp_v)

        lane = lax.iota(jnp.int32, L)
        zero16 = _splat_i32(0)
        pids = (lane, lane + L)
        pclamp = (lane, jnp.minimum(lane + L, P - 1))
        padmask = (None, pids[1] < P)

        acc = [jnp.zeros((L,), jnp.float32) for _ in range(2)]
        cnt = [jnp.zeros((L,), jnp.float32) for _ in range(2)]
        for j in range(J):
            for h in range(2):
                base = pclamp[h] * (J * 2) + (j * 2)
                idx = plsc.load_gather(kp_v, [base])
                flag = plsc.load_gather(kp_v, [base + 1])
                ok = flag == 1
                if padmask[h] is not None:
                    ok = ok & padmask[h]
                fm = jnp.where(ok, 1.0, 0.0)
                vt = plsc.load_gather(
                    tags_v, [jnp.minimum(jnp.maximum(idx, 0), N - 1)]
                )
                vt_v[j, pl.ds(h * L, L)] = vt
                fm_v[j, pl.ds(h * L, L)] = fm
                acc[h] = acc[h] + vt * fm
                cnt[h] = cnt[h] + fm

        m = []
        safe = []
        validf = []
        for h in range(2):
            sc = jnp.maximum(cnt[h], 1.0)
            vf = jnp.where(cnt[h] > 0.0, 1.0, 0.0)
            mh = vf * (acc[h] / sc)
            m_v[pl.ds(h * L, L)] = mh
            m.append(mh)
            safe.append(sc)
            validf.append(vf)
        curv = jnp.full((L,), jnp.sum(validf[0] + validf[1]), dtype=jnp.float32)
        lanef = (lane.astype(jnp.float32), (lane + L).astype(jnp.float32))
        wcur = tuple(jnp.where(lf < curv, 1.0, 0.0) for lf in lanef)

        pull_parts = []
        for h in range(2):
            pp = jnp.zeros((L,), jnp.float32)
            for j in range(J):
                d = vt_v[j, pl.ds(h * L, L)] - m[h]
                pp = pp + d * d * fm_v[j, pl.ds(h * L, L)]
            pull_parts.append(validf[h] * (pp / safe[h]))
        pullv = jnp.full(
            (L,), jnp.sum(pull_parts[0] + pull_parts[1]), dtype=jnp.float32
        )
        pullv = jnp.where(curv > 0.0, pullv / curv, pullv)

        s = [jnp.zeros((L,), jnp.float32) for _ in range(2)]
        for i in range(P - 1):
            mi = plsc.load_gather(m_v, [_splat_i32(i)])
            for h in range(2):
                hi = (h + 1) * L - 1
                if hi <= i:
                    continue
                d = mi - m[h]
                e = jnp.exp(-(d * d))
                if h * L > i:
                    s[h] = s[h] + e
                else:
                    s[h] = s[h] + jnp.where(pids[h] > i, e, 0.0)
        pushv = jnp.full(
            (L,), jnp.sum(s[0] * wcur[0] + s[1] * wcur[1]), dtype=jnp.float32
        )
        denomv = jnp.maximum(curv * (curv - 1.0), 1.0) * 0.5
        pushv = jnp.where(curv > 1.0, pushv / denomv, pushv) * 0.5

        o_v[...] = jnp.where(
            lane == 0, pushv, jnp.where(lane == 1, pullv, 0.0)
        ).astype(jnp.float32)
        pltpu.sync_copy(o_v, out_hbm.at[b])


def kernel(inp, input1):
    tags = inp.reshape(B, N)
    kp = jnp.pad(input1.reshape(B, P * J * 2), ((0, 0), (0, 4)))
    out = _aeloss_sc(tags, kp)
    return out[:, :2]
